# baseline (speedup 1.0000x reference)
import functools

import jax
import jax.numpy as jnp
from jax import lax
from jax.experimental import pallas as pl
from jax.experimental.pallas import tpu as pltpu
from jax.experimental.pallas import tpu_sc as plsc

MU = 3.5
BU = 0.1
BI = -0.05
B_UI = MU + BU + BI

N_USER = 10000
N_ITEM = 1000
H = 64
W = 128
BATCH = 16384

NUM_CORES = 2
NUM_SUBCORES = 16
L = 16
NW = NUM_CORES * NUM_SUBCORES
BPW = BATCH // NW
CHUNK = 256


def _body(u_hbm, i_hbm, r_hbm, w_hbm, p_hbm, q_hbm, out_hbm,
          u_v, i_v, f_v, p_v, q_v, w_v, r_v, o_v, sem):
    wid = lax.axis_index("s") * NUM_CORES + lax.axis_index("c")
    base = wid * BPW

    pltpu.sync_copy(u_hbm.at[pl.ds(base, BPW)], u_v)
    pltpu.sync_copy(i_hbm.at[pl.ds(base, BPW)], i_v)
    pltpu.sync_copy(r_hbm.at[pl.ds(base, BPW)], r_v)

    def flat_body(g, carry):
        s = pl.ds(g * L, L)
        f_v[s] = u_v[s] * N_ITEM + i_v[s]
        return carry

    lax.fori_loop(0, BPW // L, flat_body, 0)
    cp_w = pltpu.async_copy(w_hbm.at[f_v], w_v, sem)

    last_lane = lax.iota(jnp.int32, L) == (L - 1)

    def chunk_body(c, carry):
        cbase = c * CHUNK
        cp_p = pltpu.async_copy(p_hbm.at[u_v.at[pl.ds(cbase, CHUNK)]], p_v,
                                sem)
        cp_q = pltpu.async_copy(q_hbm.at[i_v.at[pl.ds(cbase, CHUNK)]], q_v,
                                sem)
        cp_p.wait()
        cp_q.wait()

        def row_body(b, carry2):
            acc = p_v[b, pl.ds(0, L)] * q_v[b, pl.ds(0, L)]
            for h in range(1, H // L):
                acc = acc + p_v[b, pl.ds(h * L, L)] * q_v[b, pl.ds(h * L, L)]
            tot = plsc.cumsum(acc)
            plsc.store_scatter(o_v, [jnp.full((L,), cbase + b, jnp.int32)],
                               tot, mask=last_lane)
            return carry2

        lax.fori_loop(0, CHUNK, row_body, 0)
        return carry

    lax.fori_loop(0, BPW // CHUNK, chunk_body, 0)
    cp_w.wait()

    def comb_body(g, carry):
        s = pl.ds(g * L, L)
        o_v[s] = o_v[s] + B_UI + w_v[s] * (r_v[s] - B_UI)
        return carry

    lax.fori_loop(0, BPW // L, comb_body, 0)
    pltpu.sync_copy(o_v, out_hbm.at[pl.ds(base, BPW)])


@jax.jit
def _svd_sc(u, i, r, w_ij, P, Q):
    w_flat = w_ij.reshape(-1)
    p2 = jnp.concatenate([P, P], axis=1)
    q2 = jnp.concatenate([Q, Q], axis=1)
    mesh = plsc.VectorSubcoreMesh(core_axis_name="c", subcore_axis_name="s")
    run = functools.partial(
        pl.kernel,
        mesh=mesh,
        compiler_params=pltpu.CompilerParams(needs_layout_passes=False),
        out_type=jax.ShapeDtypeStruct((BATCH,), jnp.float32),
        scratch_types=[
            pltpu.VMEM((BPW,), jnp.int32),
            pltpu.VMEM((BPW,), jnp.int32),
            pltpu.VMEM((BPW,), jnp.int32),
            pltpu.VMEM((CHUNK, W), jnp.float32),
            pltpu.VMEM((CHUNK, W), jnp.float32),
            pltpu.VMEM((BPW,), jnp.float32),
            pltpu.VMEM((BPW,), jnp.float32),
            pltpu.VMEM((BPW,), jnp.float32),
            pltpu.SemaphoreType.DMA,
        ],
    )(_body)
    return run(u, i, r, w_flat, p2, q2)


def kernel(u, i, r, w_ij, P, Q):
    u = u.astype(jnp.int32)
    i = i.astype(jnp.int32)
    return _svd_sc(u, i, r, w_ij, P, Q)


# speedup vs baseline: 1.9299x; 1.9299x over previous
import functools

import jax
import jax.numpy as jnp
from jax import lax
from jax.experimental import pallas as pl
from jax.experimental.pallas import tpu as pltpu
from jax.experimental.pallas import tpu_sc as plsc

MU = 3.5
BU = 0.1
BI = -0.05
B_UI = MU + BU + BI

N_USER = 10000
N_ITEM = 1000
H = 64
W = 128
BATCH = 16384

NUM_CORES = 2
NUM_SUBCORES = 16
L = 16
NW = NUM_CORES * NUM_SUBCORES
BPW = BATCH // NW
CHUNK = 256


def _body(u_hbm, i_hbm, r_hbm, w_hbm, wt_hbm, p_hbm, q_hbm, out_hbm,
          u_v, i_v, f_v, p_v, q_v, w_v, r_v, o_v, t_v, sem):
    wid = lax.axis_index("s") * NUM_CORES + lax.axis_index("c")
    base = wid * BPW

    pltpu.sync_copy(u_hbm.at[pl.ds(base, BPW)], u_v)
    pltpu.sync_copy(i_hbm.at[pl.ds(base, BPW)], i_v)
    pltpu.sync_copy(r_hbm.at[pl.ds(base, BPW)], r_v)

    def flat_body(g, carry):
        s = pl.ds(g * L, L)
        f_v[s] = u_v[s] * N_ITEM + i_v[s]
        return carry

    lax.fori_loop(0, BPW // L, flat_body, 0)
    cp_w = pltpu.async_copy(w_hbm.at[f_v], w_v, sem)

    cp_t = pltpu.async_copy(
        wt_hbm.at[u_v.at[pl.ds(0, CHUNK)], pl.ds(128, 128)], t_v, sem)
    cp_t.wait()

    last_lane = lax.iota(jnp.int32, L) == (L - 1)

    def chunk_body(c, carry):
        cbase = c * CHUNK
        cp_p = pltpu.async_copy(p_hbm.at[u_v.at[pl.ds(cbase, CHUNK)]], p_v,
                                sem)
        cp_q = pltpu.async_copy(q_hbm.at[i_v.at[pl.ds(cbase, CHUNK)]], q_v,
                                sem)
        cp_p.wait()
        cp_q.wait()

        def row_body(b, carry2):
            acc = p_v[b, pl.ds(0, L)] * q_v[b, pl.ds(0, L)]
            for h in range(1, H // L):
                acc = acc + p_v[b, pl.ds(h * L, L)] * q_v[b, pl.ds(h * L, L)]
            tot = plsc.cumsum(acc)
            plsc.store_scatter(o_v, [jnp.full((L,), cbase + b, jnp.int32)],
                               tot, mask=last_lane)
            return carry2

        lax.fori_loop(0, CHUNK, row_body, 0)
        return carry

    lax.fori_loop(0, BPW // CHUNK, chunk_body, 0)
    cp_w.wait()

    def comb_body(g, carry):
        s = pl.ds(g * L, L)
        o_v[s] = o_v[s] + B_UI + w_v[s] * (r_v[s] - B_UI)
        return carry

    lax.fori_loop(0, BPW // L, comb_body, 0)
    pltpu.sync_copy(o_v, out_hbm.at[pl.ds(base, BPW)])


@jax.jit
def _svd_sc(u, i, r, w_ij, P, Q):
    w_flat = w_ij.reshape(-1)
    p2 = jnp.concatenate([P, P], axis=1)
    q2 = jnp.concatenate([Q, Q], axis=1)
    mesh = plsc.VectorSubcoreMesh(core_axis_name="c", subcore_axis_name="s")
    run = functools.partial(
        pl.kernel,
        mesh=mesh,
        compiler_params=pltpu.CompilerParams(needs_layout_passes=False),
        out_type=jax.ShapeDtypeStruct((BATCH,), jnp.float32),
        scratch_types=[
            pltpu.VMEM((BPW,), jnp.int32),
            pltpu.VMEM((BPW,), jnp.int32),
            pltpu.VMEM((BPW,), jnp.int32),
            pltpu.VMEM((CHUNK, W), jnp.float32),
            pltpu.VMEM((CHUNK, W), jnp.float32),
            pltpu.VMEM((BPW,), jnp.float32),
            pltpu.VMEM((BPW,), jnp.float32),
            pltpu.VMEM((BPW,), jnp.float32),
            pltpu.VMEM((CHUNK, W), jnp.float32),
            pltpu.SemaphoreType.DMA,
        ],
    )(_body)
    return run(u, i, r, w_flat, w_ij, p2, q2)


def kernel(u, i, r, w_ij, P, Q):
    u = u.astype(jnp.int32)
    i = i.astype(jnp.int32)
    return _svd_sc(u, i, r, w_ij, P, Q)
